# baseline (speedup 1.0000x reference)
import functools

import jax
import jax.numpy as jnp
from jax import lax
from jax.experimental import pallas as pl
from jax.experimental.pallas import tpu as pltpu
from jax.experimental.pallas import tpu_sc as plsc

N_NODES = 10000
CH = 128
N_EDGES = 320000

NUM_CORES = 2
NUM_SUBCORES = 16
NUM_WORKERS = NUM_CORES * NUM_SUBCORES

CHUNK = 128
NBUF = 2
CHUNKS_PER_TILE = 80
HALF = CHUNKS_PER_TILE // 2
E_PAD = NUM_WORKERS * CHUNKS_PER_TILE * CHUNK
N_PAD = 10240
ROWS_PER_TILE = N_PAD // NUM_SUBCORES


def _sc_mesh():
    return plsc.VectorSubcoreMesh(core_axis_name="c", subcore_axis_name="s")


@functools.partial(
    pl.kernel,
    out_type=jax.ShapeDtypeStruct((NUM_CORES, N_PAD, CH), jnp.float32),
    mesh=_sc_mesh(),
    scratch_types=[
        pltpu.VMEM((CHUNKS_PER_TILE, CHUNK), jnp.int32),
        pltpu.VMEM((CHUNK, CH), jnp.float32),
        pltpu.VMEM_SHARED((N_PAD, CH), jnp.float32),
    ],
)
def _sc_degree(dst_hbm, ones_hbm, zeros_hbm, out_hbm, dst_v, ones_v, acc_sh):
    c = lax.axis_index("c")
    s = lax.axis_index("s")
    wid = s * NUM_CORES + c
    row0 = s * ROWS_PER_TILE
    pltpu.sync_copy(zeros_hbm.at[pl.ds(row0, ROWS_PER_TILE)],
                    acc_sh.at[pl.ds(row0, ROWS_PER_TILE)])
    pltpu.sync_copy(ones_hbm, ones_v)
    pltpu.sync_copy(dst_hbm.at[wid], dst_v)
    plsc.subcore_barrier()

    def body(j, carry):
        pltpu.sync_copy(ones_v, acc_sh.at[dst_v.at[j]], add=True)
        return carry

    lax.fori_loop(0, CHUNKS_PER_TILE, body, 0, unroll=False)
    plsc.subcore_barrier()
    pltpu.sync_copy(acc_sh.at[pl.ds(row0, ROWS_PER_TILE)],
                    out_hbm.at[c, pl.ds(row0, ROWS_PER_TILE)])


@functools.partial(
    pl.kernel,
    out_type=jax.ShapeDtypeStruct((NUM_CORES, N_PAD, CH), jnp.float32),
    mesh=_sc_mesh(),
    scratch_types=[
        pltpu.VMEM((HALF, CHUNK), jnp.int32),
        pltpu.VMEM((HALF, CHUNK), jnp.int32),
        pltpu.VMEM((NBUF, CHUNK, CH), jnp.float32),
        pltpu.VMEM_SHARED((N_PAD, CH), jnp.float32),
        pltpu.SemaphoreType.DMA((NBUF,)),
        pltpu.SemaphoreType.DMA((NBUF,)),
    ],
)
def _sc_edge_pass(y_hbm, src_hbm, dst_hbm, zeros_hbm, out_hbm,
                  src_v, dst_v, rows_v, acc_sh, gsem, ssem):
    c = lax.axis_index("c")
    s = lax.axis_index("s")
    wid = s * NUM_CORES + c
    row0 = s * ROWS_PER_TILE
    pltpu.sync_copy(zeros_hbm.at[pl.ds(row0, ROWS_PER_TILE)],
                    acc_sh.at[pl.ds(row0, ROWS_PER_TILE)])
    plsc.subcore_barrier()

    def run_half(base):
        pltpu.sync_copy(src_hbm.at[wid, pl.ds(base, HALF)], src_v)
        pltpu.sync_copy(dst_hbm.at[wid, pl.ds(base, HALF)], dst_v)
        for b in range(NBUF):
            pltpu.async_copy(y_hbm.at[src_v.at[b]], rows_v.at[b],
                             gsem.at[b])

        def body(k, carry):
            sdesc = []
            for b in range(NBUF):
                j = k * NBUF + b
                pltpu.make_async_copy(
                    y_hbm.at[src_v.at[j]], rows_v.at[b], gsem.at[b]).wait()
                sdesc.append(pltpu.async_copy(
                    rows_v.at[b], acc_sh.at[dst_v.at[j]], ssem.at[b],
                    add=True))
            for b in range(NBUF):
                j = k * NBUF + b
                sdesc[b].wait()

                @pl.when(j + NBUF < HALF)
                def _():
                    pltpu.async_copy(y_hbm.at[src_v.at[j + NBUF]],
                                     rows_v.at[b], gsem.at[b])
            return carry

        lax.fori_loop(0, HALF // NBUF, body, 0, unroll=False)

    run_half(0)
    run_half(HALF)
    plsc.subcore_barrier()
    pltpu.sync_copy(acc_sh.at[pl.ds(row0, ROWS_PER_TILE)],
                    out_hbm.at[c, pl.ds(row0, ROWS_PER_TILE)])


def _tc1_body(h_ref, t_ref, wt1_ref, bt1_ref, wt2_ref, bt2_ref, wc1_ref,
              deg_ref, y1_ref, dinv_ref):
    degp = deg_ref[...]
    deg = degp[0, :N_NODES, 0:1] + degp[1, :N_NODES, 0:1] + 1.0
    dinv = lax.rsqrt(deg)
    tt = t_ref[0, 0]
    e1 = jnp.maximum(tt * wt1_ref[...] + bt1_ref[...], 0.0)
    temb = jnp.dot(e1, wt2_ref[...],
                   preferred_element_type=jnp.float32) + bt2_ref[...]
    h = h_ref[...] + temb
    x1 = jnp.dot(h, wc1_ref[...], preferred_element_type=jnp.float32)
    y1_ref[...] = x1 * dinv
    dinv_ref[...] = dinv


def _tc2_body(s1_ref, y1_ref, dinv_ref, bc1_ref, lng_ref, lnb_ref, wc2_ref,
              y2_ref):
    sp = s1_ref[...]
    ssum = sp[0, :N_NODES, :] + sp[1, :N_NODES, :]
    dinv = dinv_ref[...]
    o1 = dinv * (ssum + y1_ref[...]) + bc1_ref[...]
    mu = jnp.mean(o1, axis=-1, keepdims=True)
    d = o1 - mu
    var = jnp.mean(d * d, axis=-1, keepdims=True)
    z = d * lax.rsqrt(var + 1e-5) * lng_ref[...] + lnb_ref[...]
    z = jnp.maximum(z, 0.0)
    x2 = jnp.dot(z, wc2_ref[...], preferred_element_type=jnp.float32)
    y2_ref[...] = x2 * dinv


def _tc3_body(s2_ref, y2_ref, dinv_ref, bc2_ref, out_ref):
    sp = s2_ref[...]
    ssum = sp[0, :N_NODES, :] + sp[1, :N_NODES, :]
    out_ref[...] = dinv_ref[...] * (ssum + y2_ref[...]) + bc2_ref[...]


def kernel(h_noisy, edge_index, t, W_t1, b_t1, W_t2, b_t2,
           W_c1, b_c1, W_c2, b_c2, ln_g, ln_b):
    f32 = jnp.float32
    src = edge_index[0].astype(jnp.int32)
    dst = edge_index[1].astype(jnp.int32)
    npad = E_PAD - N_EDGES
    pad_ar = jnp.arange(npad, dtype=jnp.int32)
    pad_src = pad_ar % N_NODES
    pad_dst = N_NODES + pad_ar % (N_PAD - N_NODES)
    src_p = jnp.concatenate([src, pad_src]).reshape(
        NUM_WORKERS, CHUNKS_PER_TILE, CHUNK)
    dst_p = jnp.concatenate([dst, pad_dst]).reshape(
        NUM_WORKERS, CHUNKS_PER_TILE, CHUNK)

    ones_ch = jnp.ones((CHUNK, CH), f32)
    zeros_ch = jnp.zeros((N_PAD, CH), f32)

    deg_part = _sc_degree(dst_p, ones_ch, zeros_ch)

    y1, dinv = pl.pallas_call(
        _tc1_body,
        out_shape=[jax.ShapeDtypeStruct((N_NODES, CH), f32),
                   jax.ShapeDtypeStruct((N_NODES, 1), f32)],
    )(h_noisy, t.reshape(1, 1).astype(f32), W_t1, b_t1.reshape(1, CH),
      W_t2, b_t2.reshape(1, CH), W_c1, deg_part)

    s1 = _sc_edge_pass(y1, src_p, dst_p, zeros_ch)

    y2 = pl.pallas_call(
        _tc2_body,
        out_shape=jax.ShapeDtypeStruct((N_NODES, CH), f32),
    )(s1, y1, dinv, b_c1.reshape(1, CH), ln_g.reshape(1, CH),
      ln_b.reshape(1, CH), W_c2)

    s2 = _sc_edge_pass(y2, src_p, dst_p, zeros_ch)

    out = pl.pallas_call(
        _tc3_body,
        out_shape=jax.ShapeDtypeStruct((N_NODES, CH), f32),
    )(s2, y2, dinv, b_c2.reshape(1, CH))
    return out


# speedup vs baseline: 1.2062x; 1.2062x over previous
import functools

import jax
import jax.numpy as jnp
from jax import lax
from jax.experimental import pallas as pl
from jax.experimental.pallas import tpu as pltpu
from jax.experimental.pallas import tpu_sc as plsc

N_NODES = 10000
CH = 128
N_EDGES = 320000

NUM_CORES = 2
NUM_SUBCORES = 16
NUM_WORKERS = NUM_CORES * NUM_SUBCORES

CHUNK = 128
NBUF = 2
CHUNKS_PER_TILE = 80
HALF = CHUNKS_PER_TILE // 2
E_PAD = NUM_WORKERS * CHUNKS_PER_TILE * CHUNK
N_PAD = 10240
ROWS_PER_TILE = N_PAD // NUM_SUBCORES


def _sc_mesh():
    return plsc.VectorSubcoreMesh(core_axis_name="c", subcore_axis_name="s")


@functools.partial(
    pl.kernel,
    out_type=jax.ShapeDtypeStruct((NUM_CORES, N_PAD, CH), jnp.float32),
    mesh=_sc_mesh(),
    scratch_types=[
        pltpu.VMEM((CHUNKS_PER_TILE, CHUNK), jnp.int32),
        pltpu.VMEM((CHUNK, CH), jnp.float32),
        pltpu.VMEM_SHARED((N_PAD, CH), jnp.float32),
    ],
)
def _sc_degree(dst_hbm, ones_hbm, zeros_hbm, out_hbm, dst_v, ones_v, acc_sh):
    c = lax.axis_index("c")
    s = lax.axis_index("s")
    wid = s * NUM_CORES + c
    row0 = s * ROWS_PER_TILE
    pltpu.sync_copy(zeros_hbm.at[pl.ds(row0, ROWS_PER_TILE)],
                    acc_sh.at[pl.ds(row0, ROWS_PER_TILE)])
    pltpu.sync_copy(ones_hbm, ones_v)
    pltpu.sync_copy(dst_hbm.at[wid], dst_v)
    plsc.subcore_barrier()

    def body(j, carry):
        pltpu.sync_copy(ones_v, acc_sh.at[dst_v.at[j]], add=True)
        return carry

    lax.fori_loop(0, CHUNKS_PER_TILE, body, 0, unroll=False)
    plsc.subcore_barrier()
    pltpu.sync_copy(acc_sh.at[pl.ds(row0, ROWS_PER_TILE)],
                    out_hbm.at[c, pl.ds(row0, ROWS_PER_TILE)])


@functools.partial(
    pl.kernel,
    out_type=jax.ShapeDtypeStruct((NUM_CORES, N_PAD, CH), jnp.float32),
    mesh=_sc_mesh(),
    scratch_types=[
        pltpu.VMEM((HALF, CHUNK), jnp.int32),
        pltpu.VMEM((HALF, CHUNK), jnp.int32),
        pltpu.VMEM((NBUF, CHUNK, CH), jnp.float32),
        pltpu.VMEM_SHARED((N_PAD, CH), jnp.float32),
        pltpu.SemaphoreType.DMA((NBUF,)),
        pltpu.SemaphoreType.DMA((NBUF,)),
    ],
)
def _sc_edge_pass(y_hbm, src_hbm, dst_hbm, zeros_hbm, out_hbm,
                  src_v, dst_v, rows_v, acc_sh, gsem, ssem):
    c = lax.axis_index("c")
    s = lax.axis_index("s")
    wid = s * NUM_CORES + c
    row0 = s * ROWS_PER_TILE
    pltpu.sync_copy(zeros_hbm.at[pl.ds(row0, ROWS_PER_TILE)],
                    acc_sh.at[pl.ds(row0, ROWS_PER_TILE)])
    plsc.subcore_barrier()

    def run_half(base):
        pltpu.sync_copy(src_hbm.at[wid, pl.ds(base, HALF)], src_v)
        pltpu.sync_copy(dst_hbm.at[wid, pl.ds(base, HALF)], dst_v)
        for b in range(NBUF):
            pltpu.async_copy(y_hbm.at[src_v.at[b]], rows_v.at[b],
                             gsem.at[b])

        def body(k, carry):
            for b in range(NBUF):
                j = k * NBUF + b
                pltpu.make_async_copy(
                    y_hbm.at[src_v.at[j]], rows_v.at[b], gsem.at[b]).wait()
                pltpu.sync_copy(rows_v.at[b], acc_sh.at[dst_v.at[j]],
                                add=True)

                @pl.when(j + NBUF < HALF)
                def _():
                    pltpu.async_copy(y_hbm.at[src_v.at[j + NBUF]],
                                     rows_v.at[b], gsem.at[b])
            return carry

        lax.fori_loop(0, HALF // NBUF, body, 0, unroll=False)

    run_half(0)
    run_half(HALF)
    plsc.subcore_barrier()
    pltpu.sync_copy(acc_sh.at[pl.ds(row0, ROWS_PER_TILE)],
                    out_hbm.at[c, pl.ds(row0, ROWS_PER_TILE)])


def _tc1_body(h_ref, t_ref, wt1_ref, bt1_ref, wt2_ref, bt2_ref, wc1_ref,
              deg_ref, y1_ref, dinv_ref):
    degp = deg_ref[...]
    deg = degp[0, :N_NODES, 0:1] + degp[1, :N_NODES, 0:1] + 1.0
    dinv = lax.rsqrt(deg)
    tt = t_ref[0, 0]
    e1 = jnp.maximum(tt * wt1_ref[...] + bt1_ref[...], 0.0)
    temb = jnp.dot(e1, wt2_ref[...],
                   preferred_element_type=jnp.float32) + bt2_ref[...]
    h = h_ref[...] + temb
    x1 = jnp.dot(h, wc1_ref[...], preferred_element_type=jnp.float32)
    y1_ref[...] = x1 * dinv
    dinv_ref[...] = dinv


def _tc2_body(s1_ref, y1_ref, dinv_ref, bc1_ref, lng_ref, lnb_ref, wc2_ref,
              y2_ref):
    sp = s1_ref[...]
    ssum = sp[0, :N_NODES, :] + sp[1, :N_NODES, :]
    dinv = dinv_ref[...]
    o1 = dinv * (ssum + y1_ref[...]) + bc1_ref[...]
    mu = jnp.mean(o1, axis=-1, keepdims=True)
    d = o1 - mu
    var = jnp.mean(d * d, axis=-1, keepdims=True)
    z = d * lax.rsqrt(var + 1e-5) * lng_ref[...] + lnb_ref[...]
    z = jnp.maximum(z, 0.0)
    x2 = jnp.dot(z, wc2_ref[...], preferred_element_type=jnp.float32)
    y2_ref[...] = x2 * dinv


def _tc3_body(s2_ref, y2_ref, dinv_ref, bc2_ref, out_ref):
    sp = s2_ref[...]
    ssum = sp[0, :N_NODES, :] + sp[1, :N_NODES, :]
    out_ref[...] = dinv_ref[...] * (ssum + y2_ref[...]) + bc2_ref[...]


def kernel(h_noisy, edge_index, t, W_t1, b_t1, W_t2, b_t2,
           W_c1, b_c1, W_c2, b_c2, ln_g, ln_b):
    f32 = jnp.float32
    src = edge_index[0].astype(jnp.int32)
    dst = edge_index[1].astype(jnp.int32)
    npad = E_PAD - N_EDGES
    pad_ar = jnp.arange(npad, dtype=jnp.int32)
    pad_src = pad_ar % N_NODES
    pad_dst = N_NODES + pad_ar % (N_PAD - N_NODES)
    src_p = jnp.concatenate([src, pad_src]).reshape(
        NUM_WORKERS, CHUNKS_PER_TILE, CHUNK)
    dst_p = jnp.concatenate([dst, pad_dst]).reshape(
        NUM_WORKERS, CHUNKS_PER_TILE, CHUNK)

    ones_ch = jnp.ones((CHUNK, CH), f32)
    zeros_ch = jnp.zeros((N_PAD, CH), f32)

    deg_part = _sc_degree(dst_p, ones_ch, zeros_ch)

    y1, dinv = pl.pallas_call(
        _tc1_body,
        out_shape=[jax.ShapeDtypeStruct((N_NODES, CH), f32),
                   jax.ShapeDtypeStruct((N_NODES, 1), f32)],
    )(h_noisy, t.reshape(1, 1).astype(f32), W_t1, b_t1.reshape(1, CH),
      W_t2, b_t2.reshape(1, CH), W_c1, deg_part)

    s1 = _sc_edge_pass(y1, src_p, dst_p, zeros_ch)

    y2 = pl.pallas_call(
        _tc2_body,
        out_shape=jax.ShapeDtypeStruct((N_NODES, CH), f32),
    )(s1, y1, dinv, b_c1.reshape(1, CH), ln_g.reshape(1, CH),
      ln_b.reshape(1, CH), W_c2)

    s2 = _sc_edge_pass(y2, src_p, dst_p, zeros_ch)

    out = pl.pallas_call(
        _tc3_body,
        out_shape=jax.ShapeDtypeStruct((N_NODES, CH), f32),
    )(s2, y2, dinv, b_c2.reshape(1, CH))
    return out


# speedup vs baseline: 1.2278x; 1.0179x over previous
import functools

import jax
import jax.numpy as jnp
from jax import lax
from jax.experimental import pallas as pl
from jax.experimental.pallas import tpu as pltpu
from jax.experimental.pallas import tpu_sc as plsc

N_NODES = 10000
CH = 128
N_EDGES = 320000

NUM_CORES = 2
NUM_SUBCORES = 16
NUM_WORKERS = NUM_CORES * NUM_SUBCORES

CHUNK = 128
NBUF = 2
CHUNKS_PER_TILE = 80
HALF = CHUNKS_PER_TILE // 2
E_PAD = NUM_WORKERS * CHUNKS_PER_TILE * CHUNK
N_PAD = 10240
ROWS_PER_TILE = N_PAD // NUM_SUBCORES


def _sc_mesh():
    return plsc.VectorSubcoreMesh(core_axis_name="c", subcore_axis_name="s")


def _fill_rows(ref, vec16):
    for i in range(CHUNK):
        for k in range(CH // 16):
            ref[i, pl.ds(k * 16, 16)] = vec16


@functools.partial(
    pl.kernel,
    out_type=jax.ShapeDtypeStruct((NUM_CORES, N_PAD, CH), jnp.float32),
    mesh=_sc_mesh(),
    scratch_types=[
        pltpu.VMEM((CHUNKS_PER_TILE, CHUNK), jnp.int32),
        pltpu.VMEM((CHUNK, CH), jnp.float32),
        pltpu.VMEM_SHARED((N_PAD, CH), jnp.float32),
    ],
)
def _sc_degree(dst_hbm, out_hbm, dst_v, ones_v, acc_sh):
    c = lax.axis_index("c")
    s = lax.axis_index("s")
    wid = s * NUM_CORES + c
    row0 = s * ROWS_PER_TILE
    _fill_rows(ones_v, jnp.zeros((16,), jnp.float32))
    for k in range(ROWS_PER_TILE // CHUNK):
        pltpu.sync_copy(ones_v, acc_sh.at[pl.ds(row0 + k * CHUNK, CHUNK)])
    _fill_rows(ones_v, jnp.ones((16,), jnp.float32))
    pltpu.sync_copy(dst_hbm.at[wid], dst_v)
    plsc.subcore_barrier()

    def body(j, carry):
        pltpu.sync_copy(ones_v, acc_sh.at[dst_v.at[j]], add=True)
        return carry

    lax.fori_loop(0, CHUNKS_PER_TILE, body, 0, unroll=False)
    plsc.subcore_barrier()
    pltpu.sync_copy(acc_sh.at[pl.ds(row0, ROWS_PER_TILE)],
                    out_hbm.at[c, pl.ds(row0, ROWS_PER_TILE)])


@functools.partial(
    pl.kernel,
    out_type=jax.ShapeDtypeStruct((NUM_CORES, N_PAD, CH), jnp.float32),
    mesh=_sc_mesh(),
    scratch_types=[
        pltpu.VMEM((HALF, CHUNK), jnp.int32),
        pltpu.VMEM((HALF, CHUNK), jnp.int32),
        pltpu.VMEM((NBUF, CHUNK, CH), jnp.float32),
        pltpu.VMEM_SHARED((N_PAD, CH), jnp.float32),
        pltpu.SemaphoreType.DMA((NBUF,)),
    ],
)
def _sc_edge_pass(y_hbm, src_hbm, dst_hbm, out_hbm,
                  src_v, dst_v, rows_v, acc_sh, gsem):
    c = lax.axis_index("c")
    s = lax.axis_index("s")
    wid = s * NUM_CORES + c
    row0 = s * ROWS_PER_TILE
    _fill_rows(rows_v.at[0], jnp.zeros((16,), jnp.float32))
    for k in range(ROWS_PER_TILE // CHUNK):
        pltpu.sync_copy(rows_v.at[0],
                        acc_sh.at[pl.ds(row0 + k * CHUNK, CHUNK)])
    plsc.subcore_barrier()

    def run_half(base):
        pltpu.sync_copy(src_hbm.at[wid, pl.ds(base, HALF)], src_v)
        pltpu.sync_copy(dst_hbm.at[wid, pl.ds(base, HALF)], dst_v)
        for b in range(NBUF):
            pltpu.async_copy(y_hbm.at[src_v.at[b]], rows_v.at[b],
                             gsem.at[b])

        def body(k, carry):
            for b in range(NBUF):
                j = k * NBUF + b
                pltpu.make_async_copy(
                    y_hbm.at[src_v.at[j]], rows_v.at[b], gsem.at[b]).wait()
                pltpu.sync_copy(rows_v.at[b], acc_sh.at[dst_v.at[j]],
                                add=True)

                @pl.when(j + NBUF < HALF)
                def _():
                    pltpu.async_copy(y_hbm.at[src_v.at[j + NBUF]],
                                     rows_v.at[b], gsem.at[b])
            return carry

        lax.fori_loop(0, HALF // NBUF, body, 0, unroll=False)

    run_half(0)
    run_half(HALF)
    plsc.subcore_barrier()
    pltpu.sync_copy(acc_sh.at[pl.ds(row0, ROWS_PER_TILE)],
                    out_hbm.at[c, pl.ds(row0, ROWS_PER_TILE)])


def _tc1_body(h_ref, t_ref, wt1_ref, bt1_ref, wt2_ref, bt2_ref, wc1_ref,
              deg_ref, y1_ref, dinv_ref):
    degp = deg_ref[...]
    deg = degp[0, :N_NODES, 0:1] + degp[1, :N_NODES, 0:1] + 1.0
    dinv = lax.rsqrt(deg)
    tt = t_ref[0, 0]
    e1 = jnp.maximum(tt * wt1_ref[...] + bt1_ref[...], 0.0)
    temb = jnp.dot(e1, wt2_ref[...],
                   preferred_element_type=jnp.float32) + bt2_ref[...]
    h = h_ref[...] + temb
    x1 = jnp.dot(h, wc1_ref[...], preferred_element_type=jnp.float32)
    y1_ref[...] = x1 * dinv
    dinv_ref[...] = dinv


def _tc2_body(s1_ref, y1_ref, dinv_ref, bc1_ref, lng_ref, lnb_ref, wc2_ref,
              y2_ref):
    sp = s1_ref[...]
    ssum = sp[0, :N_NODES, :] + sp[1, :N_NODES, :]
    dinv = dinv_ref[...]
    o1 = dinv * (ssum + y1_ref[...]) + bc1_ref[...]
    mu = jnp.mean(o1, axis=-1, keepdims=True)
    d = o1 - mu
    var = jnp.mean(d * d, axis=-1, keepdims=True)
    z = d * lax.rsqrt(var + 1e-5) * lng_ref[...] + lnb_ref[...]
    z = jnp.maximum(z, 0.0)
    x2 = jnp.dot(z, wc2_ref[...], preferred_element_type=jnp.float32)
    y2_ref[...] = x2 * dinv


def _tc3_body(s2_ref, y2_ref, dinv_ref, bc2_ref, out_ref):
    sp = s2_ref[...]
    ssum = sp[0, :N_NODES, :] + sp[1, :N_NODES, :]
    out_ref[...] = dinv_ref[...] * (ssum + y2_ref[...]) + bc2_ref[...]


def kernel(h_noisy, edge_index, t, W_t1, b_t1, W_t2, b_t2,
           W_c1, b_c1, W_c2, b_c2, ln_g, ln_b):
    f32 = jnp.float32
    src = edge_index[0].astype(jnp.int32)
    dst = edge_index[1].astype(jnp.int32)
    npad = E_PAD - N_EDGES
    pad_ar = jnp.arange(npad, dtype=jnp.int32)
    pad_src = pad_ar % N_NODES
    pad_dst = N_NODES + pad_ar % (N_PAD - N_NODES)
    src_p = jnp.concatenate([src, pad_src]).reshape(
        NUM_WORKERS, CHUNKS_PER_TILE, CHUNK)
    dst_p = jnp.concatenate([dst, pad_dst]).reshape(
        NUM_WORKERS, CHUNKS_PER_TILE, CHUNK)

    deg_part = _sc_degree(dst_p)

    y1, dinv = pl.pallas_call(
        _tc1_body,
        out_shape=[jax.ShapeDtypeStruct((N_NODES, CH), f32),
                   jax.ShapeDtypeStruct((N_NODES, 1), f32)],
    )(h_noisy, t.reshape(1, 1).astype(f32), W_t1, b_t1.reshape(1, CH),
      W_t2, b_t2.reshape(1, CH), W_c1, deg_part)

    s1 = _sc_edge_pass(y1, src_p, dst_p)

    y2 = pl.pallas_call(
        _tc2_body,
        out_shape=jax.ShapeDtypeStruct((N_NODES, CH), f32),
    )(s1, y1, dinv, b_c1.reshape(1, CH), ln_g.reshape(1, CH),
      ln_b.reshape(1, CH), W_c2)

    s2 = _sc_edge_pass(y2, src_p, dst_p)

    out = pl.pallas_call(
        _tc3_body,
        out_shape=jax.ShapeDtypeStruct((N_NODES, CH), f32),
    )(s2, y2, dinv, b_c2.reshape(1, CH))
    return out


# speedup vs baseline: 1.2299x; 1.0017x over previous
import functools

import jax
import jax.numpy as jnp
from jax import lax
from jax.experimental import pallas as pl
from jax.experimental.pallas import tpu as pltpu
from jax.experimental.pallas import tpu_sc as plsc

N_NODES = 10000
CH = 128
N_EDGES = 320000

NUM_CORES = 2
NUM_SUBCORES = 16
NUM_WORKERS = NUM_CORES * NUM_SUBCORES

CHUNK = 128
NBUF = 2
CHUNKS_PER_TILE = 80
HALF = CHUNKS_PER_TILE // 2
E_PAD = NUM_WORKERS * CHUNKS_PER_TILE * CHUNK
N_PAD = 10240
ROWS_PER_TILE = N_PAD // NUM_SUBCORES


def _sc_mesh():
    return plsc.VectorSubcoreMesh(core_axis_name="c", subcore_axis_name="s")


def _fill_rows(ref, vec16):
    for i in range(CHUNK):
        for k in range(CH // 16):
            ref[i, pl.ds(k * 16, 16)] = vec16


@functools.partial(
    pl.kernel,
    out_type=jax.ShapeDtypeStruct((NUM_CORES, N_PAD, CH), jnp.float32),
    mesh=_sc_mesh(),
    scratch_types=[
        pltpu.VMEM((CHUNKS_PER_TILE, CHUNK), jnp.int32),
        pltpu.VMEM((CHUNK, CH), jnp.float32),
        pltpu.VMEM_SHARED((N_PAD, CH), jnp.float32),
    ],
)
def _sc_degree(dst_hbm, out_hbm, dst_v, ones_v, acc_sh):
    c = lax.axis_index("c")
    s = lax.axis_index("s")
    wid = s * NUM_CORES + c
    row0 = s * ROWS_PER_TILE
    _fill_rows(ones_v, jnp.zeros((16,), jnp.float32))
    for k in range(ROWS_PER_TILE // CHUNK):
        pltpu.sync_copy(ones_v, acc_sh.at[pl.ds(row0 + k * CHUNK, CHUNK)])
    _fill_rows(ones_v, jnp.ones((16,), jnp.float32))
    pltpu.sync_copy(dst_hbm.at[wid], dst_v)
    plsc.subcore_barrier()

    def body(j, carry):
        pltpu.sync_copy(ones_v, acc_sh.at[dst_v.at[j]], add=True)
        return carry

    lax.fori_loop(0, CHUNKS_PER_TILE, body, 0, unroll=False)
    plsc.subcore_barrier()
    pltpu.sync_copy(acc_sh.at[pl.ds(row0, ROWS_PER_TILE)],
                    out_hbm.at[c, pl.ds(row0, ROWS_PER_TILE)])


@functools.partial(
    pl.kernel,
    out_type=jax.ShapeDtypeStruct((NUM_CORES, N_PAD, CH), jnp.float32),
    mesh=_sc_mesh(),
    scratch_types=[
        pltpu.VMEM((HALF, CHUNK), jnp.int32),
        pltpu.VMEM((HALF, CHUNK), jnp.int32),
        pltpu.VMEM((NBUF, CHUNK, CH), jnp.float32),
        pltpu.VMEM_SHARED((N_PAD, CH), jnp.float32),
        pltpu.SemaphoreType.DMA((NBUF,)),
    ],
)
def _sc_edge_pass(y_hbm, src_hbm, dst_hbm, out_hbm,
                  src_v, dst_v, rows_v, acc_sh, gsem):
    c = lax.axis_index("c")
    s = lax.axis_index("s")
    wid = s * NUM_CORES + c
    row0 = s * ROWS_PER_TILE
    _fill_rows(rows_v.at[0], jnp.zeros((16,), jnp.float32))
    for k in range(ROWS_PER_TILE // CHUNK):
        pltpu.sync_copy(rows_v.at[0],
                        acc_sh.at[pl.ds(row0 + k * CHUNK, CHUNK)])
    plsc.subcore_barrier()

    def run_half(base):
        pltpu.sync_copy(src_hbm.at[wid, pl.ds(base, HALF)], src_v)
        pltpu.sync_copy(dst_hbm.at[wid, pl.ds(base, HALF)], dst_v)
        for b in range(NBUF):
            pltpu.async_copy(y_hbm.at[src_v.at[b]], rows_v.at[b],
                             gsem.at[b])

        def body(k, carry):
            for b in range(NBUF):
                j = k * NBUF + b
                pltpu.make_async_copy(
                    y_hbm.at[src_v.at[j]], rows_v.at[b], gsem.at[b]).wait()
                pltpu.sync_copy(rows_v.at[b], acc_sh.at[dst_v.at[j]],
                                add=True)

                @pl.when(j + NBUF < HALF)
                def _():
                    pltpu.async_copy(y_hbm.at[src_v.at[j + NBUF]],
                                     rows_v.at[b], gsem.at[b])
            return carry

        lax.fori_loop(0, HALF // NBUF, body, 0, unroll=2)

    run_half(0)
    run_half(HALF)
    plsc.subcore_barrier()
    pltpu.sync_copy(acc_sh.at[pl.ds(row0, ROWS_PER_TILE)],
                    out_hbm.at[c, pl.ds(row0, ROWS_PER_TILE)])


def _tc1_body(h_ref, t_ref, wt1_ref, bt1_ref, wt2_ref, bt2_ref, wc1_ref,
              deg_ref, y1_ref, dinv_ref):
    degp = deg_ref[...]
    deg = degp[0, :N_NODES, 0:1] + degp[1, :N_NODES, 0:1] + 1.0
    dinv = lax.rsqrt(deg)
    tt = t_ref[0, 0]
    e1 = jnp.maximum(tt * wt1_ref[...] + bt1_ref[...], 0.0)
    temb = jnp.dot(e1, wt2_ref[...],
                   preferred_element_type=jnp.float32) + bt2_ref[...]
    h = h_ref[...] + temb
    x1 = jnp.dot(h, wc1_ref[...], preferred_element_type=jnp.float32)
    y1_ref[...] = x1 * dinv
    dinv_ref[...] = dinv


def _tc2_body(s1_ref, y1_ref, dinv_ref, bc1_ref, lng_ref, lnb_ref, wc2_ref,
              y2_ref):
    sp = s1_ref[...]
    ssum = sp[0, :N_NODES, :] + sp[1, :N_NODES, :]
    dinv = dinv_ref[...]
    o1 = dinv * (ssum + y1_ref[...]) + bc1_ref[...]
    mu = jnp.mean(o1, axis=-1, keepdims=True)
    d = o1 - mu
    var = jnp.mean(d * d, axis=-1, keepdims=True)
    z = d * lax.rsqrt(var + 1e-5) * lng_ref[...] + lnb_ref[...]
    z = jnp.maximum(z, 0.0)
    x2 = jnp.dot(z, wc2_ref[...], preferred_element_type=jnp.float32)
    y2_ref[...] = x2 * dinv


def _tc3_body(s2_ref, y2_ref, dinv_ref, bc2_ref, out_ref):
    sp = s2_ref[...]
    ssum = sp[0, :N_NODES, :] + sp[1, :N_NODES, :]
    out_ref[...] = dinv_ref[...] * (ssum + y2_ref[...]) + bc2_ref[...]


def kernel(h_noisy, edge_index, t, W_t1, b_t1, W_t2, b_t2,
           W_c1, b_c1, W_c2, b_c2, ln_g, ln_b):
    f32 = jnp.float32
    src = edge_index[0].astype(jnp.int32)
    dst = edge_index[1].astype(jnp.int32)
    npad = E_PAD - N_EDGES
    pad_ar = jnp.arange(npad, dtype=jnp.int32)
    pad_src = pad_ar % N_NODES
    pad_dst = N_NODES + pad_ar % (N_PAD - N_NODES)
    src_p = jnp.concatenate([src, pad_src]).reshape(
        NUM_WORKERS, CHUNKS_PER_TILE, CHUNK)
    dst_p = jnp.concatenate([dst, pad_dst]).reshape(
        NUM_WORKERS, CHUNKS_PER_TILE, CHUNK)

    deg_part = _sc_degree(dst_p)

    y1, dinv = pl.pallas_call(
        _tc1_body,
        out_shape=[jax.ShapeDtypeStruct((N_NODES, CH), f32),
                   jax.ShapeDtypeStruct((N_NODES, 1), f32)],
    )(h_noisy, t.reshape(1, 1).astype(f32), W_t1, b_t1.reshape(1, CH),
      W_t2, b_t2.reshape(1, CH), W_c1, deg_part)

    s1 = _sc_edge_pass(y1, src_p, dst_p)

    y2 = pl.pallas_call(
        _tc2_body,
        out_shape=jax.ShapeDtypeStruct((N_NODES, CH), f32),
    )(s1, y1, dinv, b_c1.reshape(1, CH), ln_g.reshape(1, CH),
      ln_b.reshape(1, CH), W_c2)

    s2 = _sc_edge_pass(y2, src_p, dst_p)

    out = pl.pallas_call(
        _tc3_body,
        out_shape=jax.ShapeDtypeStruct((N_NODES, CH), f32),
    )(s2, y2, dinv, b_c2.reshape(1, CH))
    return out
